# baseline (speedup 1.0000x reference)
import jax
import jax.numpy as jnp
from jax import lax
from jax.experimental import pallas as pl
from jax.experimental.pallas import tpu as pltpu

N_EXPERTS = 8
TOP_K = 2
INPUT_DIM = 267
HIDDEN = 1024
N_LAYERS = 4
N_CLASSES = 5
BATCH = 256

PAD_IN = 384
PAD_C = 128
T_ROWS = 64


def _layernorm(h, s, b):
    mu = jnp.mean(h, axis=-1, keepdims=True)
    var = jnp.mean((h - mu) * (h - mu), axis=-1, keepdims=True)
    return (h - mu) * jax.lax.rsqrt(var + 1e-5) * s + b


def _router_kernel(xT_ref, Wr_ref, br_ref, combT_ref, cnt_ref):
    logitsT = jax.lax.dot_general(
        Wr_ref[...], xT_ref[...],
        dimension_numbers=(((0,), (0,)), ((), ())),
        preferred_element_type=jnp.float32)
    logitsT = logitsT + br_ref[:, 0:1]
    m = jnp.max(logitsT, axis=0, keepdims=True)
    p = jnp.exp(logitsT - m)
    probs = p / jnp.sum(p, axis=0, keepdims=True)
    iota = jax.lax.broadcasted_iota(jnp.int32, probs.shape, 0)
    v1 = jnp.max(probs, axis=0, keepdims=True)
    i1 = jnp.min(jnp.where(probs == v1, iota, N_EXPERTS),
                 axis=0, keepdims=True)
    oh1 = (iota == i1).astype(jnp.float32)
    masked = jnp.where(oh1 > 0, -jnp.inf, probs)
    v2 = jnp.max(masked, axis=0, keepdims=True)
    i2 = jnp.min(jnp.where(masked == v2, iota, N_EXPERTS),
                 axis=0, keepdims=True)
    oh2 = (iota == i2).astype(jnp.float32)
    combT = (v1 * oh1 + v2 * oh2) / (v1 + v2)
    combT_ref[...] = combT
    cnt = jnp.sum((combT > 0).astype(jnp.int32), axis=1, keepdims=True)
    cnt_ref[...] = jnp.broadcast_to(cnt, (N_EXPERTS, 128))


def _router(xT, Wr_p, br_2d):
    return pl.pallas_call(
        _router_kernel,
        out_shape=(jax.ShapeDtypeStruct((N_EXPERTS, BATCH), jnp.float32),
                   jax.ShapeDtypeStruct((N_EXPERTS, 128), jnp.int32)),
    )(xT, Wr_p, br_2d)


def _mlp_kernel(cnt_ref, x_ref, combT_ref, W_in_ref, b_in_ref, ln_s_ref,
                ln_b_ref, W_h_ref, b_h_ref, cls_s_ref, cls_b_ref, W_out_ref,
                b_out_ref, out_ref, o_acc):
    e = pl.program_id(0)

    @pl.when(e == 0)
    def _init():
        out_ref[...] = jnp.zeros_like(out_ref)

    o_acc[...] = jnp.zeros_like(o_acc)

    comb_row = combT_ref[pl.ds(e, 1), :]
    mask = (comb_row > 0).astype(jnp.float32)
    tri_r = jax.lax.broadcasted_iota(jnp.int32, (BATCH, BATCH), 0)
    tri_c = jax.lax.broadcasted_iota(jnp.int32, (BATCH, BATCH), 1)
    U = (tri_r <= tri_c).astype(jnp.float32)
    rank = jnp.dot(mask, U, preferred_element_type=jnp.float32) - 1.0

    cnt = cnt_ref[e]
    n_tiles = (cnt + T_ROWS - 1) // T_ROWS

    def body(i, carry):
        t0 = i * T_ROWS
        r_sub = jax.lax.broadcasted_iota(jnp.int32, (T_ROWS, BATCH), 0) + t0
        P = jnp.where((rank == r_sub.astype(jnp.float32)) & (mask > 0),
                      1.0, 0.0)
        xt = jnp.dot(P, x_ref[...], preferred_element_type=jnp.float32)

        h = jnp.dot(xt, W_in_ref[0],
                    preferred_element_type=jnp.float32) + b_in_ref[0, 0]
        h = jax.nn.gelu(h)
        for l in range(N_LAYERS):
            hn = _layernorm(h, ln_s_ref[0, l], ln_b_ref[0, l])
            h = h + jax.nn.gelu(
                jnp.dot(hn, W_h_ref[0, l], preferred_element_type=jnp.float32)
                + b_h_ref[0, l])
        hn = _layernorm(h, cls_s_ref[0, 0], cls_b_ref[0, 0])
        o = jnp.dot(hn, W_out_ref[0], preferred_element_type=jnp.float32) \
            + b_out_ref[0, 0]
        o_acc[pl.ds(t0, T_ROWS), :] = o
        return carry

    lax.fori_loop(0, n_tiles, body, 0)

    r_full = jax.lax.broadcasted_iota(jnp.int32, (BATCH, BATCH), 0)
    Pw = jnp.where((rank == r_full.astype(jnp.float32)) & (mask > 0),
                   comb_row, 0.0)
    out_ref[...] += jax.lax.dot_general(
        Pw, o_acc[...],
        dimension_numbers=(((0,), (0,)), ((), ())),
        preferred_element_type=jnp.float32)


def _mlp(counts, x_p, combT, W_in_p, b_in_3, ln_s, ln_b, W_h, b_h,
         cls_s_3, cls_b_3, W_out_p, b_out_3):
    full = lambda *shape: pl.BlockSpec(shape, lambda e, c: (0,) * len(shape))
    per_e = lambda *shape: pl.BlockSpec((1,) + shape,
                                        lambda e, c: (e,) + (0,) * len(shape))
    grid_spec = pltpu.PrefetchScalarGridSpec(
        num_scalar_prefetch=1,
        grid=(N_EXPERTS,),
        in_specs=[
            full(BATCH, PAD_IN),
            full(N_EXPERTS, BATCH),
            per_e(PAD_IN, HIDDEN),
            per_e(1, HIDDEN),
            per_e(N_LAYERS, HIDDEN),
            per_e(N_LAYERS, HIDDEN),
            per_e(N_LAYERS, HIDDEN, HIDDEN),
            per_e(N_LAYERS, HIDDEN),
            per_e(1, HIDDEN),
            per_e(1, HIDDEN),
            per_e(HIDDEN, PAD_C),
            per_e(1, PAD_C),
        ],
        out_specs=pl.BlockSpec((BATCH, PAD_C), lambda e, c: (0, 0)),
        scratch_shapes=[pltpu.VMEM((BATCH, PAD_C), jnp.float32)],
    )
    out = pl.pallas_call(
        _mlp_kernel,
        grid_spec=grid_spec,
        out_shape=jax.ShapeDtypeStruct((BATCH, PAD_C), jnp.float32),
        compiler_params=pltpu.CompilerParams(
            dimension_semantics=("arbitrary",)),
    )(counts, x_p, combT, W_in_p, b_in_3, ln_s, ln_b, W_h, b_h,
      cls_s_3, cls_b_3, W_out_p, b_out_3)
    return out[:, :N_CLASSES]


def kernel(x, Wr, br, W_in, b_in, ln_s, ln_b, W_h, b_h,
           cls_ln_s, cls_ln_b, W_out, b_out):
    x_p = jnp.pad(x, ((0, 0), (0, PAD_IN - INPUT_DIM)))
    xT = x_p.T
    Wr_p = jnp.pad(Wr, ((0, PAD_IN - INPUT_DIM), (0, 0)))
    br_2d = jnp.broadcast_to(br.reshape(N_EXPERTS, 1), (N_EXPERTS, 128))
    W_in_p = jnp.pad(W_in, ((0, 0), (0, PAD_IN - INPUT_DIM), (0, 0)))
    W_out_p = jnp.pad(W_out, ((0, 0), (0, 0), (0, PAD_C - N_CLASSES)))
    b_out_p = jnp.pad(b_out, ((0, 0), (0, PAD_C - N_CLASSES)))
    b_in_3 = b_in.reshape(N_EXPERTS, 1, HIDDEN)
    cls_s_3 = cls_ln_s.reshape(N_EXPERTS, 1, HIDDEN)
    cls_b_3 = cls_ln_b.reshape(N_EXPERTS, 1, HIDDEN)
    b_out_3 = b_out_p.reshape(N_EXPERTS, 1, PAD_C)

    combT, cnt = _router(xT, Wr_p, br_2d)
    counts = cnt[:, 0]

    return _mlp(counts, x_p, combT, W_in_p, b_in_3, ln_s, ln_b, W_h, b_h,
                cls_s_3, cls_b_3, W_out_p, b_out_3)


# speedup vs baseline: 1.0229x; 1.0229x over previous
import jax
import jax.numpy as jnp
from jax import lax
from jax.experimental import pallas as pl
from jax.experimental.pallas import tpu as pltpu

N_EXPERTS = 8
TOP_K = 2
INPUT_DIM = 267
HIDDEN = 1024
N_LAYERS = 4
N_CLASSES = 5
BATCH = 256

PAD_IN = 272
PAD_C = 128
T_ROWS = 64


def _layernorm(h, s, b):
    mu = jnp.mean(h, axis=-1, keepdims=True)
    var = jnp.mean((h - mu) * (h - mu), axis=-1, keepdims=True)
    return (h - mu) * jax.lax.rsqrt(var + 1e-5) * s + b


def _router_kernel(xT_ref, Wr_ref, br_ref, combT_ref, cnt_ref, rank_ref):
    logitsT = jax.lax.dot_general(
        Wr_ref[...], xT_ref[...],
        dimension_numbers=(((0,), (0,)), ((), ())),
        preferred_element_type=jnp.float32)
    logitsT = logitsT + br_ref[:, 0:1]
    m = jnp.max(logitsT, axis=0, keepdims=True)
    p = jnp.exp(logitsT - m)
    probs = p / jnp.sum(p, axis=0, keepdims=True)
    iota = jax.lax.broadcasted_iota(jnp.int32, probs.shape, 0)
    v1 = jnp.max(probs, axis=0, keepdims=True)
    i1 = jnp.min(jnp.where(probs == v1, iota, N_EXPERTS),
                 axis=0, keepdims=True)
    oh1 = (iota == i1).astype(jnp.float32)
    masked = jnp.where(oh1 > 0, -jnp.inf, probs)
    v2 = jnp.max(masked, axis=0, keepdims=True)
    i2 = jnp.min(jnp.where(masked == v2, iota, N_EXPERTS),
                 axis=0, keepdims=True)
    oh2 = (iota == i2).astype(jnp.float32)
    combT = (v1 * oh1 + v2 * oh2) / (v1 + v2)
    combT_ref[...] = combT
    cnt = jnp.sum((combT > 0).astype(jnp.int32), axis=1, keepdims=True)
    cnt_ref[...] = jnp.broadcast_to(cnt, (N_EXPERTS, 128))
    maskT = (combT > 0).astype(jnp.float32)
    tri_r = jax.lax.broadcasted_iota(jnp.int32, (BATCH, BATCH), 0)
    tri_c = jax.lax.broadcasted_iota(jnp.int32, (BATCH, BATCH), 1)
    U = (tri_r <= tri_c).astype(jnp.float32)
    rank_ref[...] = jnp.dot(maskT, U,
                            preferred_element_type=jnp.float32) - 1.0


def _router(xT, Wr_p, br_2d):
    return pl.pallas_call(
        _router_kernel,
        out_shape=(jax.ShapeDtypeStruct((N_EXPERTS, BATCH), jnp.float32),
                   jax.ShapeDtypeStruct((N_EXPERTS, 128), jnp.int32),
                   jax.ShapeDtypeStruct((N_EXPERTS, BATCH), jnp.float32)),
    )(xT, Wr_p, br_2d)


def _mlp_kernel(cnt_ref, x_ref, combT_ref, rank_ref, W_in_ref, b_in_ref,
                ln_s_ref, ln_b_ref, W_h_ref, b_h_ref, cls_s_ref, cls_b_ref,
                W_out_ref, b_out_ref, out_ref, o_acc):
    e = pl.program_id(0)

    @pl.when(e == 0)
    def _init():
        out_ref[...] = jnp.zeros_like(out_ref)
        o_acc[...] = jnp.zeros_like(o_acc)

    comb_row = combT_ref[pl.ds(e, 1), :]
    mask = comb_row > 0
    rank = rank_ref[pl.ds(e, 1), :]

    cnt = cnt_ref[e]
    n_tiles = (cnt + T_ROWS - 1) // T_ROWS

    def body(i, carry):
        t0 = i * T_ROWS
        r_sub = jax.lax.broadcasted_iota(jnp.int32, (T_ROWS, BATCH), 0) + t0
        P = jnp.where((rank == r_sub.astype(jnp.float32)) & mask,
                      1.0, 0.0)
        xt = jnp.dot(P, x_ref[...], preferred_element_type=jnp.float32)

        h = jnp.dot(xt, W_in_ref[0],
                    preferred_element_type=jnp.float32) + b_in_ref[0, 0]
        h = jax.nn.gelu(h)
        for l in range(N_LAYERS):
            hn = _layernorm(h, ln_s_ref[0, l], ln_b_ref[0, l])
            h = h + jax.nn.gelu(
                jnp.dot(hn, W_h_ref[0, l], preferred_element_type=jnp.float32)
                + b_h_ref[0, l])
        hn = _layernorm(h, cls_s_ref[0, 0], cls_b_ref[0, 0])
        o = jnp.dot(hn, W_out_ref[0], preferred_element_type=jnp.float32) \
            + b_out_ref[0, 0]
        o_acc[pl.ds(t0, T_ROWS), :] = o
        return carry

    lax.fori_loop(0, n_tiles, body, 0)

    r_full = jax.lax.broadcasted_iota(jnp.int32, (BATCH, BATCH), 0)
    Pw = jnp.where((rank == r_full.astype(jnp.float32)) & mask,
                   comb_row, 0.0)
    out_ref[...] += jax.lax.dot_general(
        Pw, o_acc[...],
        dimension_numbers=(((0,), (0,)), ((), ())),
        preferred_element_type=jnp.float32)


def _mlp(counts, x_p, combT, rankT, W_in_p, b_in_3, ln_s, ln_b, W_h, b_h,
         cls_s_3, cls_b_3, W_out_p, b_out_3):
    full = lambda *shape: pl.BlockSpec(shape, lambda e, c: (0,) * len(shape))
    per_e = lambda *shape: pl.BlockSpec((1,) + shape,
                                        lambda e, c: (e,) + (0,) * len(shape))
    grid_spec = pltpu.PrefetchScalarGridSpec(
        num_scalar_prefetch=1,
        grid=(N_EXPERTS,),
        in_specs=[
            full(BATCH, PAD_IN),
            full(N_EXPERTS, BATCH),
            full(N_EXPERTS, BATCH),
            per_e(PAD_IN, HIDDEN),
            per_e(1, HIDDEN),
            per_e(N_LAYERS, HIDDEN),
            per_e(N_LAYERS, HIDDEN),
            per_e(N_LAYERS, HIDDEN, HIDDEN),
            per_e(N_LAYERS, HIDDEN),
            per_e(1, HIDDEN),
            per_e(1, HIDDEN),
            per_e(HIDDEN, PAD_C),
            per_e(1, PAD_C),
        ],
        out_specs=pl.BlockSpec((BATCH, PAD_C), lambda e, c: (0, 0)),
        scratch_shapes=[pltpu.VMEM((BATCH, PAD_C), jnp.float32)],
    )
    out = pl.pallas_call(
        _mlp_kernel,
        grid_spec=grid_spec,
        out_shape=jax.ShapeDtypeStruct((BATCH, PAD_C), jnp.float32),
        compiler_params=pltpu.CompilerParams(
            dimension_semantics=("arbitrary",)),
    )(counts, x_p, combT, rankT, W_in_p, b_in_3, ln_s, ln_b, W_h, b_h,
      cls_s_3, cls_b_3, W_out_p, b_out_3)
    return out[:, :N_CLASSES]


def kernel(x, Wr, br, W_in, b_in, ln_s, ln_b, W_h, b_h,
           cls_ln_s, cls_ln_b, W_out, b_out):
    x_p = jnp.pad(x, ((0, 0), (0, PAD_IN - INPUT_DIM)))
    xT = x_p.T
    Wr_p = jnp.pad(Wr, ((0, PAD_IN - INPUT_DIM), (0, 0)))
    br_2d = jnp.broadcast_to(br.reshape(N_EXPERTS, 1), (N_EXPERTS, 128))
    W_in_p = jnp.pad(W_in, ((0, 0), (0, PAD_IN - INPUT_DIM), (0, 0)))
    W_out_p = jnp.pad(W_out, ((0, 0), (0, 0), (0, PAD_C - N_CLASSES)))
    b_out_p = jnp.pad(b_out, ((0, 0), (0, PAD_C - N_CLASSES)))
    b_in_3 = b_in.reshape(N_EXPERTS, 1, HIDDEN)
    cls_s_3 = cls_ln_s.reshape(N_EXPERTS, 1, HIDDEN)
    cls_b_3 = cls_ln_b.reshape(N_EXPERTS, 1, HIDDEN)
    b_out_3 = b_out_p.reshape(N_EXPERTS, 1, PAD_C)

    combT, cnt, rankT = _router(xT, Wr_p, br_2d)
    counts = cnt[:, 0]

    return _mlp(counts, x_p, combT, rankT, W_in_p, b_in_3, ln_s, ln_b,
                W_h, b_h, cls_s_3, cls_b_3, W_out_p, b_out_3)


# speedup vs baseline: 1.0927x; 1.0683x over previous
import jax
import jax.numpy as jnp
from jax.experimental import pallas as pl
from jax.experimental.pallas import tpu as pltpu

N_EXPERTS = 8
TOP_K = 2
INPUT_DIM = 267
HIDDEN = 1024
N_LAYERS = 4
N_CLASSES = 5
BATCH = 256

PAD_IN = 272
PAD_C = 128


def _layernorm(h, s, b):
    mu = jnp.mean(h, axis=-1, keepdims=True)
    var = jnp.mean((h - mu) * (h - mu), axis=-1, keepdims=True)
    return (h - mu) * jax.lax.rsqrt(var + 1e-5) * s + b


def _moe_kernel(x_ref, Wr_ref, br_ref, W_in_ref, b_in_ref, ln_s_ref, ln_b_ref,
                W_h_ref, b_h_ref, cls_s_ref, cls_b_ref, W_out_ref, b_out_ref,
                out_ref, comb_ref):
    e = pl.program_id(0)

    @pl.when(e == 0)
    def _router():
        logits = jnp.dot(x_ref[...], Wr_ref[...],
                         preferred_element_type=jnp.float32) + br_ref[...]
        probs = jax.nn.softmax(logits, axis=-1)
        iota = jax.lax.broadcasted_iota(jnp.int32, probs.shape, 1)
        v1 = jnp.max(probs, axis=-1, keepdims=True)
        i1 = jnp.min(jnp.where(probs == v1, iota, N_EXPERTS),
                     axis=-1, keepdims=True)
        oh1 = (iota == i1).astype(jnp.float32)
        masked = jnp.where(oh1 > 0, -jnp.inf, probs)
        v2 = jnp.max(masked, axis=-1, keepdims=True)
        i2 = jnp.min(jnp.where(masked == v2, iota, N_EXPERTS),
                     axis=-1, keepdims=True)
        oh2 = (iota == i2).astype(jnp.float32)
        comb_ref[...] = (v1 * oh1 + v2 * oh2) / (v1 + v2)

    h = jnp.dot(x_ref[...], W_in_ref[0],
                preferred_element_type=jnp.float32) + b_in_ref[0, 0]
    h = jax.nn.gelu(h)
    for l in range(N_LAYERS):
        hn = _layernorm(h, ln_s_ref[0, l], ln_b_ref[0, l])
        h = h + jax.nn.gelu(
            jnp.dot(hn, W_h_ref[0, l], preferred_element_type=jnp.float32)
            + b_h_ref[0, l])
    hn = _layernorm(h, cls_s_ref[0, 0], cls_b_ref[0, 0])
    o = jnp.dot(hn, W_out_ref[0], preferred_element_type=jnp.float32) \
        + b_out_ref[0, 0]

    lane = jax.lax.broadcasted_iota(jnp.int32, (BATCH, N_EXPERTS), 1)
    we = jnp.sum(comb_ref[...] * (lane == e).astype(jnp.float32),
                 axis=-1, keepdims=True)
    contrib = we * o

    @pl.when(e == 0)
    def _init():
        out_ref[...] = contrib

    @pl.when(e > 0)
    def _acc():
        out_ref[...] += contrib


def _forward(x, Wr, br, W_in, b_in, ln_s, ln_b, W_h, b_h,
             cls_ln_s, cls_ln_b, W_out, b_out, interpret=False):
    x_p = jnp.pad(x, ((0, 0), (0, PAD_IN - INPUT_DIM)))
    Wr_p = jnp.pad(Wr, ((0, PAD_IN - INPUT_DIM), (0, 0)))
    W_in_p = jnp.pad(W_in, ((0, 0), (0, PAD_IN - INPUT_DIM), (0, 0)))
    W_out_p = jnp.pad(W_out, ((0, 0), (0, 0), (0, PAD_C - N_CLASSES)))
    b_out_p = jnp.pad(b_out, ((0, 0), (0, PAD_C - N_CLASSES)))
    br_p = br.reshape(1, N_EXPERTS)
    b_in_3 = b_in.reshape(N_EXPERTS, 1, HIDDEN)
    cls_s_3 = cls_ln_s.reshape(N_EXPERTS, 1, HIDDEN)
    cls_b_3 = cls_ln_b.reshape(N_EXPERTS, 1, HIDDEN)
    b_out_3 = b_out_p.reshape(N_EXPERTS, 1, PAD_C)

    full = lambda *shape: pl.BlockSpec(shape, lambda e: (0,) * len(shape))
    per_e = lambda *shape: pl.BlockSpec((1,) + shape,
                                        lambda e: (e,) + (0,) * len(shape))

    out = pl.pallas_call(
        _moe_kernel,
        grid=(N_EXPERTS,),
        in_specs=[
            full(BATCH, PAD_IN),
            full(PAD_IN, N_EXPERTS),
            full(1, N_EXPERTS),
            per_e(PAD_IN, HIDDEN),
            per_e(1, HIDDEN),
            per_e(N_LAYERS, HIDDEN),
            per_e(N_LAYERS, HIDDEN),
            per_e(N_LAYERS, HIDDEN, HIDDEN),
            per_e(N_LAYERS, HIDDEN),
            per_e(1, HIDDEN),
            per_e(1, HIDDEN),
            per_e(HIDDEN, PAD_C),
            per_e(1, PAD_C),
        ],
        out_specs=pl.BlockSpec((BATCH, PAD_C), lambda e: (0, 0)),
        out_shape=jax.ShapeDtypeStruct((BATCH, PAD_C), jnp.float32),
        scratch_shapes=[pltpu.VMEM((BATCH, N_EXPERTS), jnp.float32)],
        compiler_params=pltpu.CompilerParams(
            dimension_semantics=("arbitrary",)),
        interpret=interpret,
    )(x_p, Wr_p, br_p, W_in_p, b_in_3, ln_s, ln_b, W_h, b_h,
      cls_s_3, cls_b_3, W_out_p, b_out_3)
    return out[:, :N_CLASSES]


def kernel(x, Wr, br, W_in, b_in, ln_s, ln_b, W_h, b_h,
           cls_ln_s, cls_ln_b, W_out, b_out):
    return _forward(x, Wr, br, W_in, b_in, ln_s, ln_b, W_h, b_h,
                    cls_ln_s, cls_ln_b, W_out, b_out)


# speedup vs baseline: 1.0949x; 1.0020x over previous
import jax
import jax.numpy as jnp
from jax.experimental import pallas as pl
from jax.experimental.pallas import tpu as pltpu

N_EXPERTS = 8
TOP_K = 2
INPUT_DIM = 267
HIDDEN = 1024
N_LAYERS = 4
N_CLASSES = 5
BATCH = 256

PAD_IN = 272
PAD_C = 8


def _layernorm(h, s, b):
    mu = jnp.mean(h, axis=-1, keepdims=True)
    var = jnp.mean((h - mu) * (h - mu), axis=-1, keepdims=True)
    return (h - mu) * jax.lax.rsqrt(var + 1e-5) * s + b


def _moe_kernel(x_ref, Wr_ref, br_ref, W_in_ref, b_in_ref, ln_s_ref, ln_b_ref,
                W_h_ref, b_h_ref, cls_s_ref, cls_b_ref, W_out_ref, b_out_ref,
                out_ref, comb_ref):
    e = pl.program_id(0)

    @pl.when(e == 0)
    def _router():
        logits = jnp.dot(x_ref[...], Wr_ref[...],
                         preferred_element_type=jnp.float32) + br_ref[...]
        probs = jax.nn.softmax(logits, axis=-1)
        iota = jax.lax.broadcasted_iota(jnp.int32, probs.shape, 1)
        v1 = jnp.max(probs, axis=-1, keepdims=True)
        i1 = jnp.min(jnp.where(probs == v1, iota, N_EXPERTS),
                     axis=-1, keepdims=True)
        oh1 = (iota == i1).astype(jnp.float32)
        masked = jnp.where(oh1 > 0, -jnp.inf, probs)
        v2 = jnp.max(masked, axis=-1, keepdims=True)
        i2 = jnp.min(jnp.where(masked == v2, iota, N_EXPERTS),
                     axis=-1, keepdims=True)
        oh2 = (iota == i2).astype(jnp.float32)
        comb_ref[...] = (v1 * oh1 + v2 * oh2) / (v1 + v2)

    h = jnp.dot(x_ref[...], W_in_ref[0],
                preferred_element_type=jnp.float32) + b_in_ref[0, 0]
    h = jax.nn.gelu(h)
    for l in range(N_LAYERS):
        hn = _layernorm(h, ln_s_ref[0, l], ln_b_ref[0, l])
        h = h + jax.nn.gelu(
            jnp.dot(hn, W_h_ref[0, l], preferred_element_type=jnp.float32)
            + b_h_ref[0, l])
    hn = _layernorm(h, cls_s_ref[0, 0], cls_b_ref[0, 0])
    o = jnp.dot(hn, W_out_ref[0], preferred_element_type=jnp.float32) \
        + b_out_ref[0, 0]

    lane = jax.lax.broadcasted_iota(jnp.int32, (BATCH, N_EXPERTS), 1)
    we = jnp.sum(comb_ref[...] * (lane == e).astype(jnp.float32),
                 axis=-1, keepdims=True)
    contrib = we * o

    @pl.when(e == 0)
    def _init():
        out_ref[...] = contrib

    @pl.when(e > 0)
    def _acc():
        out_ref[...] += contrib


def _forward(x, Wr, br, W_in, b_in, ln_s, ln_b, W_h, b_h,
             cls_ln_s, cls_ln_b, W_out, b_out, interpret=False):
    x_p = jnp.pad(x, ((0, 0), (0, PAD_IN - INPUT_DIM)))
    Wr_p = jnp.pad(Wr, ((0, PAD_IN - INPUT_DIM), (0, 0)))
    W_in_p = jnp.pad(W_in, ((0, 0), (0, PAD_IN - INPUT_DIM), (0, 0)))
    W_out_p = jnp.pad(W_out, ((0, 0), (0, 0), (0, PAD_C - N_CLASSES)))
    b_out_p = jnp.pad(b_out, ((0, 0), (0, PAD_C - N_CLASSES)))
    br_p = br.reshape(1, N_EXPERTS)
    b_in_3 = b_in.reshape(N_EXPERTS, 1, HIDDEN)
    cls_s_3 = cls_ln_s.reshape(N_EXPERTS, 1, HIDDEN)
    cls_b_3 = cls_ln_b.reshape(N_EXPERTS, 1, HIDDEN)
    b_out_3 = b_out_p.reshape(N_EXPERTS, 1, PAD_C)

    full = lambda *shape: pl.BlockSpec(shape, lambda e: (0,) * len(shape))
    per_e = lambda *shape: pl.BlockSpec((1,) + shape,
                                        lambda e: (e,) + (0,) * len(shape))

    out = pl.pallas_call(
        _moe_kernel,
        grid=(N_EXPERTS,),
        in_specs=[
            full(BATCH, PAD_IN),
            full(PAD_IN, N_EXPERTS),
            full(1, N_EXPERTS),
            per_e(PAD_IN, HIDDEN),
            per_e(1, HIDDEN),
            per_e(N_LAYERS, HIDDEN),
            per_e(N_LAYERS, HIDDEN),
            per_e(N_LAYERS, HIDDEN, HIDDEN),
            per_e(N_LAYERS, HIDDEN),
            per_e(1, HIDDEN),
            per_e(1, HIDDEN),
            per_e(HIDDEN, PAD_C),
            per_e(1, PAD_C),
        ],
        out_specs=pl.BlockSpec((BATCH, PAD_C), lambda e: (0, 0)),
        out_shape=jax.ShapeDtypeStruct((BATCH, PAD_C), jnp.float32),
        scratch_shapes=[pltpu.VMEM((BATCH, N_EXPERTS), jnp.float32)],
        compiler_params=pltpu.CompilerParams(
            dimension_semantics=("arbitrary",)),
        interpret=interpret,
    )(x_p, Wr_p, br_p, W_in_p, b_in_3, ln_s, ln_b, W_h, b_h,
      cls_s_3, cls_b_3, W_out_p, b_out_3)
    return out[:, :N_CLASSES]


def kernel(x, Wr, br, W_in, b_in, ln_s, ln_b, W_h, b_h,
           cls_ln_s, cls_ln_b, W_out, b_out):
    return _forward(x, Wr, br, W_in, b_in, ln_s, ln_b, W_h, b_h,
                    cls_ln_s, cls_ln_b, W_out, b_out)
